# baseline (speedup 1.0000x reference)
import functools
import math

import jax
import jax.numpy as jnp
from jax import lax
from jax.experimental import pallas as pl
from jax.experimental.pallas import tpu as pltpu
from jax.experimental.pallas import tpu_sc as plsc

D_MODEL = 64
SCALE = math.sqrt(D_MODEL)

NC = 2
NS = 16
NW = NC * NS
LANES = 16

G = 128


@functools.lru_cache(maxsize=None)
def _make_kernel(B, V, D):
    assert B % (NW * G) == 0
    n_chunks = B // (NW * G)
    chunk_vecs = G * D // LANES

    mesh = plsc.VectorSubcoreMesh(core_axis_name="c", subcore_axis_name="s")

    @functools.partial(
        pl.kernel,
        mesh=mesh,
        out_type=jax.ShapeDtypeStruct((B, D), jnp.float32),
        scratch_types=[
            pltpu.VMEM((n_chunks, G), jnp.int32),
            pltpu.VMEM((G, D), jnp.float32),
            pltpu.SemaphoreType.DMA,
        ],
        compiler_params=pltpu.CompilerParams(use_tc_tiling_on_sc=False),
    )
    def k(x_hbm, lut_hbm, out_hbm, idx_v, rows_v, sem):
        wid = lax.axis_index("s") * NC + lax.axis_index("c")
        pltpu.sync_copy(x_hbm.at[wid], idx_v)
        base = wid * (n_chunks * G)

        def chunk(j, _):
            pltpu.async_copy(lut_hbm.at[idx_v.at[j]], rows_v, sem).wait()

            def scale(i, _):
                r = i // (D // LANES)
                c = (i % (D // LANES)) * LANES
                rows_v[r, pl.ds(c, LANES)] = rows_v[r, pl.ds(c, LANES)] * SCALE
                return 0

            lax.fori_loop(0, chunk_vecs, scale, 0, unroll=4)
            pltpu.sync_copy(rows_v, out_hbm.at[pl.ds(base + j * G, G)])
            return 0

        lax.fori_loop(0, n_chunks, chunk, 0)

    return k


def kernel(x, lut):
    B = x.size
    V, D = lut.shape
    xf = x.reshape(NW, B // (NW * G), G).astype(jnp.int32)
    out = _make_kernel(B, V, D)(xf, lut)
    return out.reshape(*x.shape, D)


# speedup vs baseline: 1.2108x; 1.2108x over previous
import functools
import math

import jax
import jax.numpy as jnp
from jax import lax
from jax.experimental import pallas as pl
from jax.experimental.pallas import tpu as pltpu
from jax.experimental.pallas import tpu_sc as plsc

D_MODEL = 64
SCALE = math.sqrt(D_MODEL)

NC = 2
NS = 16
NW = NC * NS
LANES = 16

G = 128
K = 2
ROWS = K * G
NB = 4
LOOK = 2


@functools.lru_cache(maxsize=None)
def _make_kernel(B, V, D):
    assert B % (NW * ROWS) == 0
    b_per_w = B // NW
    n_chunks = b_per_w // ROWS
    n_g = n_chunks * K
    assert n_chunks % NB == 0 and n_chunks >= 2 * NB

    mesh = plsc.VectorSubcoreMesh(core_axis_name="c", subcore_axis_name="s")

    @functools.partial(
        pl.kernel,
        mesh=mesh,
        out_type=jax.ShapeDtypeStruct((B, D), jnp.float32),
        scratch_types=[
            pltpu.VMEM((n_g, G), jnp.int32),
            pltpu.VMEM((NB, ROWS, D), jnp.float32),
            pltpu.SemaphoreType.DMA((NB,)),
            pltpu.SemaphoreType.DMA((NB,)),
        ],
        compiler_params=pltpu.CompilerParams(use_tc_tiling_on_sc=False),
    )
    def k(x_hbm, lut_hbm, out_hbm, idx_v, rows_v, gsem, ssem):
        wid = lax.axis_index("s") * NC + lax.axis_index("c")
        pltpu.sync_copy(x_hbm.at[wid], idx_v)
        base = wid * b_per_w

        def g_copies(j, b):
            return [
                pltpu.make_async_copy(
                    lut_hbm.at[idx_v.at[j * K + u]],
                    rows_v.at[b, pl.ds(u * G, G)],
                    gsem.at[b],
                )
                for u in range(K)
            ]

        def s_copy(j, b):
            return pltpu.make_async_copy(
                rows_v.at[b],
                out_hbm.at[pl.ds(base + j * ROWS, ROWS)],
                ssem.at[b],
            )

        def fire_gather(j, b):
            for cp in g_copies(j, b):
                cp.start()

        def wait_gather(j, b):
            for cp in g_copies(j, b):
                cp.wait()

        def scale_buf(b):
            def body(r, _):
                for u in range(D // LANES):
                    c = u * LANES
                    rows_v[b, r, pl.ds(c, LANES)] = (
                        rows_v[b, r, pl.ds(c, LANES)] * SCALE
                    )
                return 0

            lax.fori_loop(0, ROWS, body, 0, unroll=4)

        def process(j, b, scatter_wait, gather_fire):
            wait_gather(j, b)
            scale_buf(b)
            s_copy(j, b).start()
            bn = (b + LOOK) % NB
            if scatter_wait:
                s_copy(0, bn).wait()
            if gather_fire:
                fire_gather(j + LOOK, bn)

        for b in range(LOOK):
            fire_gather(b, b)
        for b in range(NB):
            process(b, b, scatter_wait=b >= NB - LOOK, gather_fire=True)

        def outer(o, _):
            j0 = o * NB
            for b in range(NB):
                process(j0 + b, b, scatter_wait=True, gather_fire=True)
            return 0

        lax.fori_loop(1, n_chunks // NB - 1, outer, 0)

        for b in range(NB):
            j = n_chunks - NB + b
            process(j, b, scatter_wait=b < NB - LOOK, gather_fire=b < NB - LOOK)

        for b in range(NB):
            s_copy(0, b).wait()

    return k


def kernel(x, lut):
    B = x.size
    V, D = lut.shape
    xf = x.reshape(NW, B // (NW * G), G).astype(jnp.int32)
    out = _make_kernel(B, V, D)(xf, lut)
    return out.reshape(*x.shape, D)


# speedup vs baseline: 1.3076x; 1.0800x over previous
import functools
import math

import jax
import jax.numpy as jnp
from jax import lax
from jax.experimental import pallas as pl
from jax.experimental.pallas import tpu as pltpu
from jax.experimental.pallas import tpu_sc as plsc

D_MODEL = 64
SCALE = math.sqrt(D_MODEL)

NC = 2
NS = 16
NW = NC * NS
LANES = 16

G = 128
NB = 4


@functools.lru_cache(maxsize=None)
def _make_kernel(BT, S, V, D):
    assert BT == NW * G and D == 64
    assert S % NB == 0 and S >= 2 * NB
    DT = D // 8

    mesh = plsc.VectorSubcoreMesh(core_axis_name="c", subcore_axis_name="s")

    @functools.partial(
        pl.kernel,
        mesh=mesh,
        out_type=jax.ShapeDtypeStruct((S, DT, NW, 8, G), jnp.float32),
        scratch_types=[
            pltpu.VMEM((S, G), jnp.int32),
            pltpu.VMEM((NB, G, D), jnp.float32),
            pltpu.VMEM((NB, D, G), jnp.float32),
            pltpu.SemaphoreType.DMA((NB,)),
            pltpu.SemaphoreType.DMA((NB,)),
        ],
        compiler_params=pltpu.CompilerParams(
            use_tc_tiling_on_sc=False, needs_layout_passes=False
        ),
    )
    def k(xt_hbm, lut_hbm, out_hbm, idx_v, gbuf, tbuf, gsem, ssem):
        wid = lax.axis_index("s") * NC + lax.axis_index("c")
        b0 = wid * G
        pltpu.sync_copy(xt_hbm.at[:, pl.ds(b0, G)], idx_v)

        iota = lax.iota(jnp.int32, 16)
        rot = [lax.rem(iota + kk, 16) for kk in range(16)]

        def g_copy(s, b):
            return pltpu.make_async_copy(
                lut_hbm.at[idx_v.at[s]], gbuf.at[b], gsem.at[b]
            )

        def s_copies(s, b):
            return [
                pltpu.make_async_copy(
                    tbuf.at[b, pl.ds(dt * 8, 8)],
                    out_hbm.at[s, dt, wid],
                    ssem.at[b],
                )
                for dt in range(DT)
            ]

        def transpose_scale(b):
            def blk(i, _):
                bb = (i // (D // 16)) * 16
                dd = (i % (D // 16)) * 16
                row = iota + bb
                for kk in range(16):
                    col = rot[kk] + dd
                    v = plsc.load_gather(gbuf.at[b], [row, col])
                    plsc.store_scatter(tbuf.at[b], [col, row], v * SCALE)
                return 0

            lax.fori_loop(0, (G // 16) * (D // 16), blk, 0)

        def process(s, b, scatter_wait, gather_fire):
            g_copy(s, b).wait()
            if scatter_wait:
                for cp in s_copies(0, b):
                    cp.wait()
            transpose_scale(b)
            for cp in s_copies(s, b):
                cp.start()
            if gather_fire:
                g_copy(s + NB, b).start()

        for b in range(NB):
            g_copy(b, b).start()
        for b in range(NB):
            process(b, b, scatter_wait=False, gather_fire=True)

        def outer(o, _):
            s0 = o * NB
            for b in range(NB):
                process(s0 + b, b, scatter_wait=True, gather_fire=True)
            return 0

        lax.fori_loop(1, S // NB - 1, outer, 0)

        for b in range(NB):
            process(S - NB + b, b, scatter_wait=True, gather_fire=False)

        for b in range(NB):
            for cp in s_copies(0, b):
                cp.wait()

    return k


def kernel(x, lut):
    BT, S = x.shape
    V, D = lut.shape
    xt = jnp.swapaxes(x, 0, 1).astype(jnp.int32)
    out5 = _make_kernel(BT, S, V, D)(xt, lut)
    return out5.transpose(2, 4, 0, 1, 3).reshape(BT, S, D)
